# Initial kernel scaffold; baseline (speedup 1.0000x reference)
import functools

import jax
import jax.numpy as jnp
from jax import lax
from jax.experimental import pallas as pl
from jax.experimental.pallas import tpu as pltpu
from jax.experimental.pallas import tpu_sc as plsc

_N = 10000
_E = 320000
_NFEAT = 128
_NHID = 128
_NCLASS = 64

_NC = 2
_NS = 16
_NW = _NC * _NS
_LANES = 16

_CHUNK = 128
_CPW = 79
_EPW = _CPW * _CHUNK
_E_PAD = _NW * _EPW
_RPS = _N // _NS
_ZR = 125

_ROWS_N = _N // 8


def _support_body(x_ref, w_ref, o_ref):
    o_ref[...] = jnp.dot(x_ref[...], w_ref[...],
                         preferred_element_type=jnp.float32)


_support_mm = pl.pallas_call(
    _support_body,
    grid=(8,),
    in_specs=[
        pl.BlockSpec((_ROWS_N, _NFEAT), lambda i: (i, 0)),
        pl.BlockSpec((_NFEAT, _NHID), lambda i: (0, 0)),
    ],
    out_specs=pl.BlockSpec((_ROWS_N, _NHID), lambda i: (i, 0)),
    out_shape=jax.ShapeDtypeStruct((_N, _NHID), jnp.float32),
)


def _head_body(p0_ref, p1_ref, bgc_ref, wh_ref, bh_ref, feat_ref, out_ref):
    feat = jnp.maximum(p0_ref[...] + p1_ref[...] + bgc_ref[...], 0.0)
    feat_ref[...] = feat
    logits = jnp.dot(feat, wh_ref[...], preferred_element_type=jnp.float32)
    out_ref[...] = jax.nn.sigmoid(logits + bh_ref[...])


_head = pl.pallas_call(
    _head_body,
    grid=(8,),
    in_specs=[
        pl.BlockSpec((_ROWS_N, _NHID), lambda i: (i, 0)),
        pl.BlockSpec((_ROWS_N, _NHID), lambda i: (i, 0)),
        pl.BlockSpec((_NHID,), lambda i: (0,)),
        pl.BlockSpec((_NHID, _NCLASS), lambda i: (0, 0)),
        pl.BlockSpec((_NCLASS,), lambda i: (0,)),
    ],
    out_specs=[
        pl.BlockSpec((_ROWS_N, _NHID), lambda i: (i, 0)),
        pl.BlockSpec((_ROWS_N, _NCLASS), lambda i: (i, 0)),
    ],
    out_shape=[
        jax.ShapeDtypeStruct((_N, _NHID), jnp.float32),
        jax.ShapeDtypeStruct((_N, _NCLASS), jnp.float32),
    ],
)


def _sc_body(support_hbm, src_hbm, dst_hbm, vals_hbm, out_hbm,
             srcv, dstv, valsv, rows, shared, sem):
    c = lax.axis_index("c")
    s = lax.axis_index("s")
    w = s * _NC + c

    @pl.loop(0, _ZR)
    def _(r):
        for j in range(_NHID // _LANES):
            rows[r, pl.ds(j * _LANES, _LANES)] = jnp.zeros(
                (_LANES,), jnp.float32)

    for t in range(_RPS // _ZR):
        pltpu.sync_copy(rows.at[pl.ds(0, _ZR)],
                        shared.at[pl.ds(s * _RPS + t * _ZR, _ZR)])
    plsc.subcore_barrier()

    wbase = w * _EPW
    wchunk = w * _CPW

    @pl.loop(0, _CPW)
    def _(k):
        base = wbase + k * _CHUNK
        pltpu.sync_copy(src_hbm.at[pl.ds(base, _CHUNK)], srcv)
        pltpu.sync_copy(dst_hbm.at[pl.ds(wchunk + k, 1)], dstv)
        pltpu.sync_copy(vals_hbm.at[pl.ds(base, _CHUNK)], valsv)
        pltpu.async_copy(support_hbm.at[srcv], rows, sem).wait()

        @pl.loop(0, _CHUNK)
        def _(i):
            v = valsv[i]
            for j in range(_NHID // _LANES):
                sl = (i, pl.ds(j * _LANES, _LANES))
                rows[sl] = rows[sl] * v

        pltpu.sync_copy(rows, shared.at[dstv.at[0]], add=True)

    plsc.subcore_barrier()
    pltpu.sync_copy(shared.at[pl.ds(s * _RPS, _RPS)],
                    out_hbm.at[c].at[pl.ds(s * _RPS, _RPS)])


_sc_spmm = pl.kernel(
    _sc_body,
    out_type=jax.ShapeDtypeStruct((_NC, _N, _NHID), jnp.float32),
    mesh=plsc.VectorSubcoreMesh(core_axis_name="c", subcore_axis_name="s"),
    scratch_types=[
        pltpu.VMEM((_CHUNK,), jnp.int32),
        pltpu.VMEM((1, _CHUNK), jnp.int32),
        pltpu.VMEM((_CHUNK,), jnp.float32),
        pltpu.VMEM((_CHUNK, _NHID), jnp.float32),
        pltpu.VMEM_SHARED((_N, _NHID), jnp.float32),
        pltpu.SemaphoreType.DMA,
    ],
)


def kernel(x, adj_indices, adj_values, W_gc, b_gc, W_hash, b_hash):
    support = _support_mm(x, W_gc)

    pad = _E_PAD - _E
    src = jnp.pad(adj_indices[0], (0, pad))
    dst = jnp.pad(adj_indices[1], (0, pad)).reshape(_NW * _CPW, _CHUNK)
    vals = jnp.pad(adj_values, (0, pad))

    partials = _sc_spmm(support, src, dst, vals)
    feature, out = _head(partials[0], partials[1], b_gc, W_hash, b_hash)
    return (feature, out)


# speedup vs baseline: 3.7399x; 3.7399x over previous
import functools

import jax
import jax.numpy as jnp
from jax import lax
from jax.experimental import pallas as pl
from jax.experimental.pallas import tpu as pltpu
from jax.experimental.pallas import tpu_sc as plsc

_N = 10000
_E = 320000
_NFEAT = 128
_NHID = 128
_NCLASS = 64

_NC = 2
_NS = 16
_NW = _NC * _NS
_LANES = 16

_CHUNK = 128
_CPW = 79
_EPW = _CPW * _CHUNK
_E_PAD = _NW * _EPW
_RPS = 632
_NA = _NS * _RPS
_RPS_LAST = _N - 15 * _RPS

_ROWS_N = _N // 10


def _support_body(x_ref, w_ref, o_ref):
    o_ref[...] = jnp.dot(x_ref[...], w_ref[...],
                         preferred_element_type=jnp.float32)


_support_mm = pl.pallas_call(
    _support_body,
    grid=(10,),
    in_specs=[
        pl.BlockSpec((_ROWS_N, _NFEAT), lambda i: (i, 0)),
        pl.BlockSpec((_NFEAT, _NHID), lambda i: (0, 0)),
    ],
    out_specs=pl.BlockSpec((_ROWS_N, _NHID), lambda i: (i, 0)),
    out_shape=jax.ShapeDtypeStruct((_N, _NHID), jnp.float32),
)


def _head_body(p0_ref, p1_ref, bgc_ref, wh_ref, bh_ref, feat_ref, out_ref):
    feat = jnp.maximum(p0_ref[...] + p1_ref[...] + bgc_ref[...], 0.0)
    feat_ref[...] = feat
    logits = jnp.dot(feat, wh_ref[...], preferred_element_type=jnp.float32)
    out_ref[...] = jax.nn.sigmoid(logits + bh_ref[...])


_head = pl.pallas_call(
    _head_body,
    grid=(10,),
    in_specs=[
        pl.BlockSpec((_ROWS_N, _NHID), lambda i: (i, 0)),
        pl.BlockSpec((_ROWS_N, _NHID), lambda i: (i, 0)),
        pl.BlockSpec((_NHID,), lambda i: (0,)),
        pl.BlockSpec((_NHID, _NCLASS), lambda i: (0, 0)),
        pl.BlockSpec((_NCLASS,), lambda i: (0,)),
    ],
    out_specs=[
        pl.BlockSpec((_ROWS_N, _NHID), lambda i: (i, 0)),
        pl.BlockSpec((_ROWS_N, _NCLASS), lambda i: (i, 0)),
    ],
    out_shape=[
        jax.ShapeDtypeStruct((_N, _NHID), jnp.float32),
        jax.ShapeDtypeStruct((_N, _NCLASS), jnp.float32),
    ],
)


def _sc_body(support_hbm, src_hbm, dst_hbm, vals_hbm, out_hbm,
             srcv, dstv, valsv, rows, shared, sem):
    c = lax.axis_index("c")
    s = lax.axis_index("s")
    w = s * _NC + c

    @pl.loop(0, _CHUNK)
    def _(r):
        for j in range(_NHID // _LANES):
            rows[r, pl.ds(j * _LANES, _LANES)] = jnp.zeros(
                (_LANES,), jnp.float32)

    for t in range(4):
        pltpu.sync_copy(rows,
                        shared.at[pl.ds(s * _RPS + t * _CHUNK, _CHUNK)])
    pltpu.sync_copy(rows.at[pl.ds(0, _RPS - 4 * _CHUNK)],
                    shared.at[pl.ds(s * _RPS + 4 * _CHUNK,
                                    _RPS - 4 * _CHUNK)])
    plsc.subcore_barrier()

    wbase = w * _EPW
    wchunk = w * _CPW

    @pl.loop(0, _CPW)
    def _(k):
        base = wbase + k * _CHUNK
        pltpu.sync_copy(src_hbm.at[pl.ds(base, _CHUNK)], srcv)
        pltpu.sync_copy(dst_hbm.at[wchunk + k], dstv)
        pltpu.sync_copy(vals_hbm.at[pl.ds(base, _CHUNK)], valsv)
        pltpu.async_copy(support_hbm.at[srcv], rows, sem).wait()

        @pl.loop(0, _CHUNK // _LANES)
        def _(g):
            vals16 = valsv[pl.ds(g * _LANES, _LANES)]
            for i in range(_LANES):
                v = vals16[i]
                r = g * _LANES + i
                for j in range(_NHID // _LANES):
                    sl = (r, pl.ds(j * _LANES, _LANES))
                    rows[sl] = rows[sl] * v

        pltpu.sync_copy(rows, shared.at[dstv.at[0]], add=True)

    plsc.subcore_barrier()

    @pl.when(s < _NS - 1)
    def _():
        pltpu.sync_copy(shared.at[pl.ds(s * _RPS, _RPS)],
                        out_hbm.at[c].at[pl.ds(s * _RPS, _RPS)])

    @pl.when(s == _NS - 1)
    def _():
        pltpu.sync_copy(shared.at[pl.ds((_NS - 1) * _RPS, _RPS_LAST)],
                        out_hbm.at[c].at[pl.ds((_NS - 1) * _RPS, _RPS_LAST)])


_sc_spmm = pl.kernel(
    _sc_body,
    out_type=jax.ShapeDtypeStruct((_NC, _N, _NHID), jnp.float32),
    mesh=plsc.VectorSubcoreMesh(core_axis_name="c", subcore_axis_name="s"),
    scratch_types=[
        pltpu.VMEM((_CHUNK,), jnp.int32),
        pltpu.VMEM((1, _CHUNK), jnp.int32),
        pltpu.VMEM((_CHUNK,), jnp.float32),
        pltpu.VMEM((_CHUNK, _NHID), jnp.float32),
        pltpu.VMEM_SHARED((_NA, _NHID), jnp.float32),
        pltpu.SemaphoreType.DMA,
    ],
)


def kernel(x, adj_indices, adj_values, W_gc, b_gc, W_hash, b_hash):
    support = _support_mm(x, W_gc)

    pad = _E_PAD - _E
    src = jnp.pad(adj_indices[0], (0, pad))
    dst = jnp.pad(adj_indices[1], (0, pad)).reshape(_NW * _CPW, 1, _CHUNK)
    vals = jnp.pad(adj_values, (0, pad))

    partials = _sc_spmm(support, src, dst, vals)
    feature, out = _head(partials[0], partials[1], b_gc, W_hash, b_hash)
    return (feature, out)
